# baseline (speedup 1.0000x reference)
import functools

import jax
import jax.numpy as jnp
from jax import lax
from jax.experimental import pallas as pl
from jax.experimental.pallas import tpu as pltpu
from jax.experimental.pallas import tpu_sc as plsc

VOCAB_N = 1000
VPAD = 1024
EMB_D = 128
BATCH_N = 16384
HIST = 200


def _make_sc_kernel():
    info = plsc.get_sparse_core_info()
    nc, ns = info.num_cores, info.num_subcores
    nw = nc * ns
    rpw = BATCH_N // nw
    vps = VPAD // ns
    mesh = plsc.VectorSubcoreMesh(core_axis_name="c", subcore_axis_name="s")

    @functools.partial(
        pl.kernel,
        mesh=mesh,
        out_type=jax.ShapeDtypeStruct((BATCH_N,), jnp.float32),
        compiler_params=pltpu.CompilerParams(needs_layout_passes=False),
        scratch_types=[
            pltpu.VMEM((rpw * HIST,), jnp.int32),
            pltpu.VMEM((vps, EMB_D), jnp.float32),
            pltpu.VMEM((1, EMB_D), jnp.float32),
            pltpu.VMEM((vps,), jnp.float32),
            pltpu.VMEM((VPAD,), jnp.float32),
            pltpu.VMEM((rpw,), jnp.float32),
            pltpu.VMEM((16,), jnp.float32),
            pltpu.VMEM_SHARED((VPAD,), jnp.float32),
            pltpu.SemaphoreType.DMA,
        ],
    )
    def sc_main(x_hbm, emb_hbm, w_hbm, b_hbm, out_hbm,
                x_v, emb_v, w_v, p_loc, p_v, out_v, b_v, p_share, sem):
        cid = lax.axis_index("c")
        sid = lax.axis_index("s")
        wid = sid * nc + cid
        rb = wid * rpw
        cp = pltpu.async_copy(x_hbm.at[pl.ds(rb * HIST, rpw * HIST)], x_v, sem)

        vb = sid * vps

        @pl.when(sid < ns - 1)
        def _():
            pltpu.sync_copy(emb_hbm.at[pl.ds(vb, vps), :], emb_v)

        @pl.when(sid == ns - 1)
        def _():
            last = VOCAB_N - (ns - 1) * vps
            pltpu.sync_copy(emb_hbm.at[pl.ds((ns - 1) * vps, last), :],
                            emb_v.at[pl.ds(0, last), :])

        pltpu.sync_copy(w_hbm, w_v)
        pltpu.sync_copy(b_hbm, b_v.at[pl.ds(0, 1)])
        lane = lax.broadcasted_iota(jnp.int32, (16,), 0)

        def pg_body(g, _):
            vrow = g * 16 + lane

            def pj_body(j, accs):
                a0, a1 = accs
                wv = w_v[0, pl.ds(j * 16, 16)]
                for t in range(16):
                    col = jnp.full((16,), j * 16 + t, jnp.int32)
                    ev = plsc.load_gather(emb_v, [vrow, col])
                    if t % 2 == 0:
                        a0 = a0 + ev * wv[t]
                    else:
                        a1 = a1 + ev * wv[t]
                return a0, a1

            zero = jnp.zeros((16,), jnp.float32)
            a0, a1 = lax.fori_loop(0, EMB_D // 16, pj_body, (zero, zero))
            p_loc[pl.ds(g * 16, 16)] = a0 + a1
            return 0

        lax.fori_loop(0, vps // 16, pg_body, 0)

        pltpu.sync_copy(p_loc, p_share.at[pl.ds(vb, vps)])
        plsc.subcore_barrier()
        pltpu.sync_copy(p_share, p_v)

        bias = b_v[...][0]
        cp.wait()

        def g_body(g, _):
            base_v = (g * 16 + lane) * HIST

            def l_body(i, accs):
                accs = list(accs)
                l0 = i * 8
                for u in range(8):
                    xv = plsc.load_gather(x_v, [base_v + (l0 + u)])
                    pv = plsc.load_gather(p_v, [xv])
                    accs[u % 4] = accs[u % 4] + pv
                return tuple(accs)

            zero = jnp.zeros((16,), jnp.float32)
            a0, a1, a2, a3 = lax.fori_loop(
                0, HIST // 8, l_body, (zero, zero, zero, zero)
            )
            z = (a0 + a1) + (a2 + a3) + bias
            out_v[pl.ds(g * 16, 16)] = 1.0 / (1.0 + jnp.exp(-z))
            return 0

        lax.fori_loop(0, rpw // 16, g_body, 0)
        pltpu.sync_copy(out_v, out_hbm.at[pl.ds(rb, rpw)])

    return sc_main


def kernel(x, emb_table, W, b):
    x_flat = x.reshape(BATCH_N * HIST)
    out = _make_sc_kernel()(x_flat, emb_table, W, b)
    return out.reshape(BATCH_N, 1)
